# Initial kernel scaffold; baseline (speedup 1.0000x reference)
import functools

import jax
import jax.numpy as jnp
from jax import lax
from jax.experimental import pallas as pl
from jax.experimental.pallas import tpu as pltpu
from jax.experimental.pallas import tpu_sc as plsc

N = 10000
E = 320000
D = 128
NC, NS = 2, 16
NW = NC * NS
EPW = E // NW
K = 80
CH = EPW // K
RPT = N // NS
ZR = 125
LANES = 16

_mesh = plsc.VectorSubcoreMesh(core_axis_name="c", subcore_axis_name="s")


@functools.partial(
    pl.kernel,
    out_type=[
        jax.ShapeDtypeStruct((NC * N, LANES), jnp.float32),
        jax.ShapeDtypeStruct((NC * N, LANES), jnp.float32),
    ],
    mesh=_mesh,
    scratch_types=[
        pltpu.VMEM((K,), jnp.int32),
        pltpu.VMEM((K,), jnp.int32),
        pltpu.VMEM((K, LANES), jnp.float32),
        pltpu.VMEM((RPT, LANES), jnp.float32),
        pltpu.VMEM_SHARED((N, LANES), jnp.float32),
        pltpu.VMEM_SHARED((N, LANES), jnp.float32),
    ],
)
def _degree_hist(src_hbm, dst_hbm, hs_hbm, hd_hbm,
                 sidx, didx, ones_v, zero_v, bins_s, bins_d):
    cid = lax.axis_index("c")
    sid = lax.axis_index("s")

    @pl.loop(0, K)
    def _(j):
        ones_v[j, :] = jnp.full((LANES,), 1.0, jnp.float32)

    @pl.loop(0, RPT)
    def _(j):
        zero_v[j, :] = jnp.zeros((LANES,), jnp.float32)

    r0 = sid * RPT
    pltpu.sync_copy(zero_v, bins_s.at[pl.ds(r0, RPT)])
    pltpu.sync_copy(zero_v, bins_d.at[pl.ds(r0, RPT)])
    plsc.subcore_barrier()

    base = (cid * NS + sid) * EPW

    @pl.loop(0, CH)
    def _(i):
        off = base + i * K
        pltpu.sync_copy(src_hbm.at[pl.ds(off, K)], sidx)
        pltpu.sync_copy(dst_hbm.at[pl.ds(off, K)], didx)
        pltpu.sync_copy(ones_v, bins_s.at[sidx], add=True)
        pltpu.sync_copy(ones_v, bins_d.at[didx], add=True)

    plsc.subcore_barrier()
    out_r0 = cid * N + r0
    pltpu.sync_copy(bins_s.at[pl.ds(r0, RPT)], hs_hbm.at[pl.ds(out_r0, RPT)])
    pltpu.sync_copy(bins_d.at[pl.ds(r0, RPT)], hd_hbm.at[pl.ds(out_r0, RPT)])


@functools.partial(
    pl.kernel,
    out_type=jax.ShapeDtypeStruct((NC * N, D), jnp.float32),
    mesh=_mesh,
    scratch_types=[
        pltpu.VMEM((K,), jnp.int32),
        pltpu.VMEM((K,), jnp.int32),
        pltpu.VMEM((K, D), jnp.float32),
        pltpu.VMEM((ZR, D), jnp.float32),
        pltpu.VMEM_SHARED((N, D), jnp.float32),
        pltpu.SemaphoreType.DMA,
    ],
)
def _message_pass(feat_hbm, src_hbm, dst_hbm, acc_hbm,
                  sidx, didx, rows, zero_v, acc_s, sem):
    cid = lax.axis_index("c")
    sid = lax.axis_index("s")

    @pl.loop(0, ZR)
    def _(j):
        @pl.loop(0, D // LANES)
        def _(q):
            zero_v[j, pl.ds(q * LANES, LANES)] = jnp.zeros((LANES,), jnp.float32)

    r0 = sid * RPT

    @pl.loop(0, RPT // ZR)
    def _(t):
        pltpu.sync_copy(zero_v, acc_s.at[pl.ds(r0 + t * ZR, ZR)])

    plsc.subcore_barrier()

    base = (cid * NS + sid) * EPW

    @pl.loop(0, CH)
    def _(i):
        off = base + i * K
        pltpu.sync_copy(src_hbm.at[pl.ds(off, K)], sidx)
        pltpu.sync_copy(dst_hbm.at[pl.ds(off, K)], didx)
        pltpu.async_copy(feat_hbm.at[sidx], rows, sem).wait()
        pltpu.sync_copy(rows, acc_s.at[didx], add=True)

    plsc.subcore_barrier()
    pltpu.sync_copy(acc_s.at[pl.ds(r0, RPT)],
                    acc_hbm.at[pl.ds(cid * N + r0, RPT)])


_ROWS = 1000


def _feat_body(x_ref, hs_ref, o_ref):
    deg = hs_ref[0] + hs_ref[1]
    scale = lax.rsqrt(jnp.maximum(deg[:, 0:1], 1.0))
    o_ref[...] = x_ref[...] * scale


_feat_call = pl.pallas_call(
    _feat_body,
    out_shape=jax.ShapeDtypeStruct((N, D), jnp.float32),
    grid=(N // _ROWS,),
    in_specs=[
        pl.BlockSpec((_ROWS, D), lambda i: (i, 0)),
        pl.BlockSpec((NC, _ROWS, LANES), lambda i: (0, i, 0)),
    ],
    out_specs=pl.BlockSpec((_ROWS, D), lambda i: (i, 0)),
)


def _out_body(acc_ref, feat_ref, hd_ref, w1_ref, w2_ref, o_ref):
    cs = acc_ref[0] + acc_ref[1]
    deg = hd_ref[0] + hd_ref[1]
    scale = lax.rsqrt(jnp.maximum(deg[:, 0:1], 1.0))
    t = jnp.dot(cs, w1_ref[...], preferred_element_type=jnp.float32)
    t = t + jnp.dot(feat_ref[...] * cs, w2_ref[...],
                    preferred_element_type=jnp.float32)
    o_ref[...] = t * scale


_out_call = pl.pallas_call(
    _out_body,
    out_shape=jax.ShapeDtypeStruct((N, D), jnp.float32),
    grid=(N // _ROWS,),
    in_specs=[
        pl.BlockSpec((NC, _ROWS, D), lambda i: (0, i, 0)),
        pl.BlockSpec((_ROWS, D), lambda i: (i, 0)),
        pl.BlockSpec((NC, _ROWS, LANES), lambda i: (0, i, 0)),
        pl.BlockSpec((D, D), lambda i: (0, 0)),
        pl.BlockSpec((D, D), lambda i: (0, 0)),
    ],
    out_specs=pl.BlockSpec((_ROWS, D), lambda i: (i, 0)),
)


def kernel(x, edge_index, W1, W2):
    ei = edge_index.astype(jnp.int32)
    src = ei[0]
    dst = ei[1]
    hs, hd = _degree_hist(src, dst)
    hs = hs.reshape(NC, N, LANES)
    hd = hd.reshape(NC, N, LANES)
    feat = _feat_call(x, hs)
    acc = _message_pass(feat, src, dst).reshape(NC, N, D)
    return _out_call(acc, feat, hd, W1, W2)


# speedup vs baseline: 4.2080x; 4.2080x over previous
import functools

import jax
import jax.numpy as jnp
from jax import lax
from jax.experimental import pallas as pl
from jax.experimental.pallas import tpu as pltpu
from jax.experimental.pallas import tpu_sc as plsc

N = 10000
E = 320000
D = 128
NC, NS = 2, 16
NW = NC * NS
EPW = E // NW
K = 80
CH = EPW // K
RPT = 624
TAIL = N - NS * RPT
ZR = 16
LANES = 16

_mesh = plsc.VectorSubcoreMesh(core_axis_name="c", subcore_axis_name="s")


@functools.partial(
    pl.kernel,
    out_type=[
        jax.ShapeDtypeStruct((NC * N, D), jnp.float32),
        jax.ShapeDtypeStruct((NC * N, D), jnp.float32),
    ],
    mesh=_mesh,
    scratch_types=[
        pltpu.VMEM((K,), jnp.int32),
        pltpu.VMEM((K, D), jnp.float32),
        pltpu.VMEM((ZR, D), jnp.float32),
        pltpu.VMEM_SHARED((N, D), jnp.float32),
    ],
)
def _degree_hist(src_hbm, dst_hbm, hs_hbm, hd_hbm,
                 eidx, ones_v, zero_v, bins):
    cid = lax.axis_index("c")
    sid = lax.axis_index("s")

    @pl.loop(0, K)
    def _(j):
        @pl.loop(0, D // LANES)
        def _(q):
            ones_v[j, pl.ds(q * LANES, LANES)] = jnp.full((LANES,), 1.0,
                                                          jnp.float32)

    @pl.loop(0, ZR)
    def _(j):
        @pl.loop(0, D // LANES)
        def _(q):
            zero_v[j, pl.ds(q * LANES, LANES)] = jnp.zeros((LANES,),
                                                           jnp.float32)

    r0 = sid * RPT
    base = (cid * NS + sid) * EPW

    def zero_my_slice():
        @pl.loop(0, RPT // ZR)
        def _(t):
            pltpu.sync_copy(zero_v, bins.at[pl.ds(r0 + t * ZR, ZR)])

        @pl.when(sid == NS - 1)
        def _():
            pltpu.sync_copy(zero_v, bins.at[pl.ds(NS * RPT, TAIL)])

    def hist_pass(idx_hbm, out_hbm):
        plsc.subcore_barrier()

        @pl.loop(0, CH)
        def _(i):
            pltpu.sync_copy(idx_hbm.at[pl.ds(base + i * K, K)], eidx)
            pltpu.sync_copy(ones_v, bins.at[eidx], add=True)

        plsc.subcore_barrier()
        pltpu.sync_copy(bins.at[pl.ds(r0, RPT)],
                        out_hbm.at[pl.ds(cid * N + r0, RPT)])

        @pl.when(sid == NS - 1)
        def _():
            pltpu.sync_copy(bins.at[pl.ds(NS * RPT, TAIL)],
                            out_hbm.at[pl.ds(cid * N + NS * RPT, TAIL)])

    zero_my_slice()
    hist_pass(src_hbm, hs_hbm)
    zero_my_slice()
    hist_pass(dst_hbm, hd_hbm)


@functools.partial(
    pl.kernel,
    out_type=jax.ShapeDtypeStruct((NC * N, D), jnp.float32),
    mesh=_mesh,
    scratch_types=[
        pltpu.VMEM((K,), jnp.int32),
        pltpu.VMEM((K,), jnp.int32),
        pltpu.VMEM((K, D), jnp.float32),
        pltpu.VMEM((ZR, D), jnp.float32),
        pltpu.VMEM_SHARED((N, D), jnp.float32),
        pltpu.SemaphoreType.DMA,
    ],
)
def _message_pass(feat_hbm, src_hbm, dst_hbm, acc_hbm,
                  sidx, didx, rows, zero_v, acc_s, sem):
    cid = lax.axis_index("c")
    sid = lax.axis_index("s")

    @pl.loop(0, ZR)
    def _(j):
        @pl.loop(0, D // LANES)
        def _(q):
            zero_v[j, pl.ds(q * LANES, LANES)] = jnp.zeros((LANES,), jnp.float32)

    r0 = sid * RPT

    @pl.loop(0, RPT // ZR)
    def _(t):
        pltpu.sync_copy(zero_v, acc_s.at[pl.ds(r0 + t * ZR, ZR)])

    @pl.when(sid == NS - 1)
    def _():
        pltpu.sync_copy(zero_v, acc_s.at[pl.ds(NS * RPT, TAIL)])

    plsc.subcore_barrier()

    base = (cid * NS + sid) * EPW

    @pl.loop(0, CH)
    def _(i):
        off = base + i * K
        pltpu.sync_copy(src_hbm.at[pl.ds(off, K)], sidx)
        pltpu.sync_copy(dst_hbm.at[pl.ds(off, K)], didx)
        pltpu.async_copy(feat_hbm.at[sidx], rows, sem).wait()
        pltpu.sync_copy(rows, acc_s.at[didx], add=True)

    plsc.subcore_barrier()
    pltpu.sync_copy(acc_s.at[pl.ds(r0, RPT)],
                    acc_hbm.at[pl.ds(cid * N + r0, RPT)])

    @pl.when(sid == NS - 1)
    def _():
        pltpu.sync_copy(acc_s.at[pl.ds(NS * RPT, TAIL)],
                        acc_hbm.at[pl.ds(cid * N + NS * RPT, TAIL)])


_ROWS = 1000


def _feat_body(x_ref, hs_ref, o_ref):
    deg = hs_ref[0] + hs_ref[1]
    scale = lax.rsqrt(jnp.maximum(deg, 1.0))
    o_ref[...] = x_ref[...] * scale


_feat_call = pl.pallas_call(
    _feat_body,
    out_shape=jax.ShapeDtypeStruct((N, D), jnp.float32),
    grid=(N // _ROWS,),
    in_specs=[
        pl.BlockSpec((_ROWS, D), lambda i: (i, 0)),
        pl.BlockSpec((NC, _ROWS, D), lambda i: (0, i, 0)),
    ],
    out_specs=pl.BlockSpec((_ROWS, D), lambda i: (i, 0)),
)


def _out_body(acc_ref, feat_ref, hd_ref, w1_ref, w2_ref, o_ref):
    cs = acc_ref[0] + acc_ref[1]
    deg = hd_ref[0] + hd_ref[1]
    scale = lax.rsqrt(jnp.maximum(deg, 1.0))
    t = jnp.dot(cs, w1_ref[...], preferred_element_type=jnp.float32)
    t = t + jnp.dot(feat_ref[...] * cs, w2_ref[...],
                    preferred_element_type=jnp.float32)
    o_ref[...] = t * scale


_out_call = pl.pallas_call(
    _out_body,
    out_shape=jax.ShapeDtypeStruct((N, D), jnp.float32),
    grid=(N // _ROWS,),
    in_specs=[
        pl.BlockSpec((NC, _ROWS, D), lambda i: (0, i, 0)),
        pl.BlockSpec((_ROWS, D), lambda i: (i, 0)),
        pl.BlockSpec((NC, _ROWS, D), lambda i: (0, i, 0)),
        pl.BlockSpec((D, D), lambda i: (0, 0)),
        pl.BlockSpec((D, D), lambda i: (0, 0)),
    ],
    out_specs=pl.BlockSpec((_ROWS, D), lambda i: (i, 0)),
)


def kernel(x, edge_index, W1, W2):
    ei = edge_index.astype(jnp.int32)
    src = ei[0]
    dst = ei[1]
    hs, hd = _degree_hist(src, dst)
    hs = hs.reshape(NC, N, D)
    hd = hd.reshape(NC, N, D)
    feat = _feat_call(x, hs)
    acc = _message_pass(feat, src, dst).reshape(NC, N, D)
    return _out_call(acc, feat, hd, W1, W2)


# speedup vs baseline: 6.5842x; 1.5647x over previous
import dataclasses
import functools

import jax
import jax.numpy as jnp
from jax import lax
from jax.experimental import pallas as pl
from jax.experimental.pallas import tpu as pltpu
from jax.experimental.pallas import tpu_sc as plsc

N = 10000
E = 320000
D = 128
NC, NS = 2, 16
NW = NC * NS
EPW = E // NW
K = 80
CH = EPW // K
RPT = 624
TAIL = N - NS * RPT
ZR = 16
LANES = 16

_mesh = plsc.VectorSubcoreMesh(core_axis_name="c", subcore_axis_name="s")
_cp = pltpu.CompilerParams()
if "needs_layout_passes" in pltpu.CompilerParams.__dataclass_fields__:
    _cp = dataclasses.replace(_cp, needs_layout_passes=False)

NB = 80
ECT = E // NS
CHH = ECT // K


@functools.partial(
    pl.kernel,
    out_type=[
        jax.ShapeDtypeStruct((N, D), jnp.float32),
        jax.ShapeDtypeStruct((N, D), jnp.float32),
    ],
    mesh=_mesh,
    compiler_params=_cp,
    scratch_types=[
        pltpu.VMEM((CHH, K), jnp.int32),
        pltpu.VMEM((NB, D), jnp.float32),
        pltpu.VMEM((NB,), jnp.int32),
        pltpu.VMEM((RPT + TAIL, D), jnp.float32),
        pltpu.VMEM_SHARED((NB, D), jnp.float32),
    ],
)
def _degree_hist(src_hbm, dst_hbm, hs_hbm, hd_hbm,
                 eidx, bins_v, idv, exp_v, bins_sh):
    cid = lax.axis_index("c")
    sid = lax.axis_index("s")

    @pl.loop(0, NB)
    def _(j):
        @pl.loop(0, D // LANES)
        def _(q):
            bins_v[j, pl.ds(q * LANES, LANES)] = jnp.zeros((LANES,),
                                                           jnp.float32)

    @pl.loop(0, NB // LANES)
    def _(j):
        idv[pl.ds(j * LANES, LANES)] = lax.iota(jnp.int32, LANES) + j * LANES

    @pl.when(sid == 0)
    def _():
        pltpu.sync_copy(bins_v, bins_sh.at[idv])

    @pl.when(cid == 0)
    def _():
        pltpu.sync_copy(src_hbm.at[sid], eidx)

    @pl.when(cid == 1)
    def _():
        pltpu.sync_copy(dst_hbm.at[sid], eidx)

    plsc.subcore_barrier()

    ones16 = jnp.full((LANES,), 1.0, jnp.float32)

    @pl.loop(0, CHH)
    def _(i):
        @pl.loop(0, K // LANES)
        def _(q):
            v = eidx[i, pl.ds(q * LANES, LANES)]
            r = lax.shift_right_logical(v, 7)
            c = lax.bitwise_and(v, 127)
            plsc.addupdate_scatter(bins_v, [r, c], ones16)

    pltpu.sync_copy(bins_v, bins_sh.at[idv], add=True)
    plsc.subcore_barrier()

    pltpu.sync_copy(bins_sh, bins_v)

    r0 = sid * RPT

    def expand(nrows):
        @pl.loop(0, nrows)
        def _(j):
            node = r0 + j
            r16 = jnp.full((LANES,), lax.shift_right_logical(node, 7),
                           jnp.int32)
            c16 = jnp.full((LANES,), lax.bitwise_and(node, 127), jnp.int32)
            cnt16 = plsc.load_gather(bins_v, [r16, c16])

            @pl.loop(0, D // LANES)
            def _(q):
                exp_v[j, pl.ds(q * LANES, LANES)] = cnt16

    out_hbm_sel = [hs_hbm, hd_hbm]

    @pl.when(sid < NS - 1)
    def _():
        expand(RPT)

    @pl.when(sid == NS - 1)
    def _():
        expand(RPT + TAIL)

    for c in range(NC):
        @pl.when(cid == c)
        def _():
            @pl.when(sid < NS - 1)
            def _():
                pltpu.sync_copy(exp_v.at[pl.ds(0, RPT)],
                                out_hbm_sel[c].at[pl.ds(r0, RPT)])

            @pl.when(sid == NS - 1)
            def _():
                pltpu.sync_copy(exp_v,
                                out_hbm_sel[c].at[pl.ds(r0, RPT + TAIL)])


@functools.partial(
    pl.kernel,
    out_type=jax.ShapeDtypeStruct((NC * N, D), jnp.float32),
    mesh=_mesh,
    scratch_types=[
        pltpu.VMEM((K,), jnp.int32),
        pltpu.VMEM((K,), jnp.int32),
        pltpu.VMEM((K, D), jnp.float32),
        pltpu.VMEM((ZR, D), jnp.float32),
        pltpu.VMEM_SHARED((N, D), jnp.float32),
        pltpu.SemaphoreType.DMA,
    ],
)
def _message_pass(feat_hbm, src_hbm, dst_hbm, acc_hbm,
                  sidx, didx, rows, zero_v, acc_s, sem):
    cid = lax.axis_index("c")
    sid = lax.axis_index("s")

    @pl.loop(0, ZR)
    def _(j):
        @pl.loop(0, D // LANES)
        def _(q):
            zero_v[j, pl.ds(q * LANES, LANES)] = jnp.zeros((LANES,), jnp.float32)

    r0 = sid * RPT

    @pl.loop(0, RPT // ZR)
    def _(t):
        pltpu.sync_copy(zero_v, acc_s.at[pl.ds(r0 + t * ZR, ZR)])

    @pl.when(sid == NS - 1)
    def _():
        pltpu.sync_copy(zero_v, acc_s.at[pl.ds(NS * RPT, TAIL)])

    plsc.subcore_barrier()

    base = (cid * NS + sid) * EPW

    @pl.loop(0, CH)
    def _(i):
        off = base + i * K
        pltpu.sync_copy(src_hbm.at[pl.ds(off, K)], sidx)
        pltpu.sync_copy(dst_hbm.at[pl.ds(off, K)], didx)
        pltpu.async_copy(feat_hbm.at[sidx], rows, sem).wait()
        pltpu.sync_copy(rows, acc_s.at[didx], add=True)

    plsc.subcore_barrier()
    pltpu.sync_copy(acc_s.at[pl.ds(r0, RPT)],
                    acc_hbm.at[pl.ds(cid * N + r0, RPT)])

    @pl.when(sid == NS - 1)
    def _():
        pltpu.sync_copy(acc_s.at[pl.ds(NS * RPT, TAIL)],
                        acc_hbm.at[pl.ds(cid * N + NS * RPT, TAIL)])


_ROWS = 1000


def _feat_body(x_ref, hs_ref, o_ref):
    deg = hs_ref[...]
    scale = lax.rsqrt(jnp.maximum(deg, 1.0))
    o_ref[...] = x_ref[...] * scale


_feat_call = pl.pallas_call(
    _feat_body,
    out_shape=jax.ShapeDtypeStruct((N, D), jnp.float32),
    grid=(N // _ROWS,),
    in_specs=[
        pl.BlockSpec((_ROWS, D), lambda i: (i, 0)),
        pl.BlockSpec((_ROWS, D), lambda i: (i, 0)),
    ],
    out_specs=pl.BlockSpec((_ROWS, D), lambda i: (i, 0)),
)


def _out_body(acc_ref, feat_ref, hd_ref, w1_ref, w2_ref, o_ref):
    cs = acc_ref[0] + acc_ref[1]
    deg = hd_ref[...]
    scale = lax.rsqrt(jnp.maximum(deg, 1.0))
    t = jnp.dot(cs, w1_ref[...], preferred_element_type=jnp.float32)
    t = t + jnp.dot(feat_ref[...] * cs, w2_ref[...],
                    preferred_element_type=jnp.float32)
    o_ref[...] = t * scale


_out_call = pl.pallas_call(
    _out_body,
    out_shape=jax.ShapeDtypeStruct((N, D), jnp.float32),
    grid=(N // _ROWS,),
    in_specs=[
        pl.BlockSpec((NC, _ROWS, D), lambda i: (0, i, 0)),
        pl.BlockSpec((_ROWS, D), lambda i: (i, 0)),
        pl.BlockSpec((_ROWS, D), lambda i: (i, 0)),
        pl.BlockSpec((D, D), lambda i: (0, 0)),
        pl.BlockSpec((D, D), lambda i: (0, 0)),
    ],
    out_specs=pl.BlockSpec((_ROWS, D), lambda i: (i, 0)),
)


def kernel(x, edge_index, W1, W2):
    ei = edge_index.astype(jnp.int32)
    src = ei[0]
    dst = ei[1]
    hs, hd = _degree_hist(src.reshape(NS, CHH, K), dst.reshape(NS, CHH, K))
    feat = _feat_call(x, hs)
    acc = _message_pass(feat, src, dst).reshape(NC, N, D)
    return _out_call(acc, feat, hd, W1, W2)


# speedup vs baseline: 12.3791x; 1.8801x over previous
import dataclasses
import functools

import jax
import jax.numpy as jnp
from jax import lax
from jax.experimental import pallas as pl
from jax.experimental.pallas import tpu as pltpu
from jax.experimental.pallas import tpu_sc as plsc

N = 10000
E = 320000
D = 128
NC, NS = 2, 16
NW = NC * NS
EPW = E // NW
K = 80
CH = EPW // K
SEG = 5
CHS = CH // SEG
RPT = 624
TAIL = N - NS * RPT
ZR = 16
LANES = 16

_mesh = plsc.VectorSubcoreMesh(core_axis_name="c", subcore_axis_name="s")
_cp = pltpu.CompilerParams()
if "needs_layout_passes" in pltpu.CompilerParams.__dataclass_fields__:
    _cp = dataclasses.replace(_cp, needs_layout_passes=False)

NB = 80
ECT = E // NS
CHH = ECT // K


@functools.partial(
    pl.kernel,
    out_type=[
        jax.ShapeDtypeStruct((N, D), jnp.float32),
        jax.ShapeDtypeStruct((N, D), jnp.float32),
    ],
    mesh=_mesh,
    compiler_params=_cp,
    scratch_types=[
        pltpu.VMEM((CHH, K), jnp.int32),
        pltpu.VMEM((NB, D), jnp.float32),
        pltpu.VMEM((NB,), jnp.int32),
        pltpu.VMEM((RPT + TAIL, D), jnp.float32),
        pltpu.VMEM_SHARED((NB, D), jnp.float32),
    ],
)
def _degree_hist(src_hbm, dst_hbm, hs_hbm, hd_hbm,
                 eidx, bins_v, idv, exp_v, bins_sh):
    cid = lax.axis_index("c")
    sid = lax.axis_index("s")

    @pl.loop(0, NB)
    def _(j):
        @pl.loop(0, D // LANES)
        def _(q):
            bins_v[j, pl.ds(q * LANES, LANES)] = jnp.zeros((LANES,),
                                                           jnp.float32)

    @pl.loop(0, NB // LANES)
    def _(j):
        idv[pl.ds(j * LANES, LANES)] = lax.iota(jnp.int32, LANES) + j * LANES

    @pl.when(sid == 0)
    def _():
        pltpu.sync_copy(bins_v, bins_sh.at[idv])

    @pl.when(cid == 0)
    def _():
        pltpu.sync_copy(src_hbm.at[sid], eidx)

    @pl.when(cid == 1)
    def _():
        pltpu.sync_copy(dst_hbm.at[sid], eidx)

    plsc.subcore_barrier()

    ones16 = jnp.full((LANES,), 1.0, jnp.float32)

    @pl.loop(0, CHH)
    def _(i):
        @pl.loop(0, K // LANES)
        def _(q):
            v = eidx[i, pl.ds(q * LANES, LANES)]
            r = lax.shift_right_logical(v, 7)
            c = lax.bitwise_and(v, 127)
            plsc.addupdate_scatter(bins_v, [r, c], ones16)

    pltpu.sync_copy(bins_v, bins_sh.at[idv], add=True)
    plsc.subcore_barrier()

    pltpu.sync_copy(bins_sh, bins_v)

    r0 = sid * RPT

    def expand(nrows):
        @pl.loop(0, nrows)
        def _(j):
            node = r0 + j
            r16 = jnp.full((LANES,), lax.shift_right_logical(node, 7),
                           jnp.int32)
            c16 = jnp.full((LANES,), lax.bitwise_and(node, 127), jnp.int32)
            cnt16 = plsc.load_gather(bins_v, [r16, c16])

            @pl.loop(0, D // LANES)
            def _(q):
                exp_v[j, pl.ds(q * LANES, LANES)] = cnt16

    out_hbm_sel = [hs_hbm, hd_hbm]

    @pl.when(sid < NS - 1)
    def _():
        expand(RPT)

    @pl.when(sid == NS - 1)
    def _():
        expand(RPT + TAIL)

    for c in range(NC):
        @pl.when(cid == c)
        def _():
            @pl.when(sid < NS - 1)
            def _():
                pltpu.sync_copy(exp_v.at[pl.ds(0, RPT)],
                                out_hbm_sel[c].at[pl.ds(r0, RPT)])

            @pl.when(sid == NS - 1)
            def _():
                pltpu.sync_copy(exp_v,
                                out_hbm_sel[c].at[pl.ds(r0, RPT + TAIL)])


@functools.partial(
    pl.kernel,
    out_type=jax.ShapeDtypeStruct((NC * N, D), jnp.float32),
    mesh=_mesh,
    scratch_types=[
        pltpu.VMEM((CHS, K), jnp.int32),
        pltpu.VMEM((CHS, K), jnp.int32),
        pltpu.VMEM((K, D), jnp.float32),
        pltpu.VMEM((K, D), jnp.float32),
        pltpu.VMEM((ZR, D), jnp.float32),
        pltpu.VMEM_SHARED((N, D), jnp.float32),
        pltpu.SemaphoreType.DMA,
        pltpu.SemaphoreType.DMA,
    ],
)
def _message_pass(feat_hbm, src_hbm, dst_hbm, acc_hbm,
                  sidx, didx, rows0, rows1, zero_v, acc_s, sem0, sem1):
    cid = lax.axis_index("c")
    sid = lax.axis_index("s")
    w = cid * NS + sid

    @pl.loop(0, ZR)
    def _(j):
        @pl.loop(0, D // LANES)
        def _(q):
            zero_v[j, pl.ds(q * LANES, LANES)] = jnp.zeros((LANES,), jnp.float32)

    r0 = sid * RPT

    @pl.loop(0, RPT // ZR)
    def _(t):
        pltpu.sync_copy(zero_v, acc_s.at[pl.ds(r0 + t * ZR, ZR)])

    @pl.when(sid == NS - 1)
    def _():
        pltpu.sync_copy(zero_v, acc_s.at[pl.ds(NS * RPT, TAIL)])

    plsc.subcore_barrier()

    @pl.loop(0, SEG)
    def _(s):
        pltpu.sync_copy(src_hbm.at[w * SEG + s], sidx)
        pltpu.sync_copy(dst_hbm.at[w * SEG + s], didx)
        pltpu.async_copy(feat_hbm.at[sidx.at[0]], rows0, sem0)

        @pl.loop(0, CHS // 2)
        def _(t):
            i0 = 2 * t
            pltpu.async_copy(feat_hbm.at[sidx.at[i0 + 1]], rows1, sem1)
            pltpu.make_async_copy(feat_hbm.at[sidx.at[0]], rows0, sem0).wait()
            pltpu.sync_copy(rows0, acc_s.at[didx.at[i0]], add=True)

            @pl.when(i0 + 2 < CHS)
            def _():
                pltpu.async_copy(feat_hbm.at[sidx.at[i0 + 2]], rows0, sem0)

            pltpu.make_async_copy(feat_hbm.at[sidx.at[0]], rows1, sem1).wait()
            pltpu.sync_copy(rows1, acc_s.at[didx.at[i0 + 1]], add=True)

        if CHS % 2:
            pltpu.make_async_copy(feat_hbm.at[sidx.at[0]], rows0, sem0).wait()
            pltpu.sync_copy(rows0, acc_s.at[didx.at[CHS - 1]], add=True)

    plsc.subcore_barrier()
    pltpu.sync_copy(acc_s.at[pl.ds(r0, RPT)],
                    acc_hbm.at[pl.ds(cid * N + r0, RPT)])

    @pl.when(sid == NS - 1)
    def _():
        pltpu.sync_copy(acc_s.at[pl.ds(NS * RPT, TAIL)],
                        acc_hbm.at[pl.ds(cid * N + NS * RPT, TAIL)])


_ROWS = 1000


def _feat_body(x_ref, hs_ref, o_ref):
    deg = hs_ref[...]
    scale = lax.rsqrt(jnp.maximum(deg, 1.0))
    o_ref[...] = x_ref[...] * scale


_feat_call = pl.pallas_call(
    _feat_body,
    out_shape=jax.ShapeDtypeStruct((N, D), jnp.float32),
    grid=(N // _ROWS,),
    in_specs=[
        pl.BlockSpec((_ROWS, D), lambda i: (i, 0)),
        pl.BlockSpec((_ROWS, D), lambda i: (i, 0)),
    ],
    out_specs=pl.BlockSpec((_ROWS, D), lambda i: (i, 0)),
)


def _out_body(acc_ref, feat_ref, hd_ref, w1_ref, w2_ref, o_ref):
    cs = acc_ref[0] + acc_ref[1]
    deg = hd_ref[...]
    scale = lax.rsqrt(jnp.maximum(deg, 1.0))
    t = jnp.dot(cs, w1_ref[...], preferred_element_type=jnp.float32)
    t = t + jnp.dot(feat_ref[...] * cs, w2_ref[...],
                    preferred_element_type=jnp.float32)
    o_ref[...] = t * scale


_out_call = pl.pallas_call(
    _out_body,
    out_shape=jax.ShapeDtypeStruct((N, D), jnp.float32),
    grid=(N // _ROWS,),
    in_specs=[
        pl.BlockSpec((NC, _ROWS, D), lambda i: (0, i, 0)),
        pl.BlockSpec((_ROWS, D), lambda i: (i, 0)),
        pl.BlockSpec((_ROWS, D), lambda i: (i, 0)),
        pl.BlockSpec((D, D), lambda i: (0, 0)),
        pl.BlockSpec((D, D), lambda i: (0, 0)),
    ],
    out_specs=pl.BlockSpec((_ROWS, D), lambda i: (i, 0)),
)


def kernel(x, edge_index, W1, W2):
    ei = edge_index.astype(jnp.int32)
    src = ei[0]
    dst = ei[1]
    hs, hd = _degree_hist(src.reshape(NS, CHH, K), dst.reshape(NS, CHH, K))
    feat = _feat_call(x, hs)
    acc = _message_pass(feat, src.reshape(NW * SEG, CHS, K),
                        dst.reshape(NW * SEG, CHS, K)).reshape(NC, N, D)
    return _out_call(acc, feat, hd, W1, W2)


# speedup vs baseline: 13.5101x; 1.0914x over previous
import dataclasses
import functools

import jax
import jax.numpy as jnp
from jax import lax
from jax.experimental import pallas as pl
from jax.experimental.pallas import tpu as pltpu
from jax.experimental.pallas import tpu_sc as plsc

N = 10000
E = 320000
D = 128
NC, NS = 2, 16
NW = NC * NS
EPW = E // NW
K = 80
CH = EPW // K
SEG = 5
CHS = CH // SEG
RPT = 624
TAIL = N - NS * RPT
ZR = 16
LANES = 16

_mesh = plsc.VectorSubcoreMesh(core_axis_name="c", subcore_axis_name="s")
_cp = pltpu.CompilerParams()
if "needs_layout_passes" in pltpu.CompilerParams.__dataclass_fields__:
    _cp = dataclasses.replace(_cp, needs_layout_passes=False)

NB = 80
ECT = E // NS
CHH = ECT // K


@functools.partial(
    pl.kernel,
    out_type=[
        jax.ShapeDtypeStruct((N, D), jnp.float32),
        jax.ShapeDtypeStruct((N, D), jnp.float32),
    ],
    mesh=_mesh,
    compiler_params=_cp,
    scratch_types=[
        pltpu.VMEM((CHH, K), jnp.int32),
        pltpu.VMEM((NB, D), jnp.float32),
        pltpu.VMEM((NB,), jnp.int32),
        pltpu.VMEM((RPT + TAIL, D), jnp.float32),
        pltpu.VMEM_SHARED((NB, D), jnp.float32),
    ],
)
def _degree_hist(src_hbm, dst_hbm, hs_hbm, hd_hbm,
                 eidx, bins_v, idv, exp_v, bins_sh):
    cid = lax.axis_index("c")
    sid = lax.axis_index("s")

    @pl.loop(0, NB)
    def _(j):
        @pl.loop(0, D // LANES)
        def _(q):
            bins_v[j, pl.ds(q * LANES, LANES)] = jnp.zeros((LANES,),
                                                           jnp.float32)

    @pl.loop(0, NB // LANES)
    def _(j):
        idv[pl.ds(j * LANES, LANES)] = lax.iota(jnp.int32, LANES) + j * LANES

    @pl.when(sid == 0)
    def _():
        pltpu.sync_copy(bins_v, bins_sh.at[idv])

    @pl.when(cid == 0)
    def _():
        pltpu.sync_copy(src_hbm.at[sid], eidx)

    @pl.when(cid == 1)
    def _():
        pltpu.sync_copy(dst_hbm.at[sid], eidx)

    plsc.subcore_barrier()

    ones16 = jnp.full((LANES,), 1.0, jnp.float32)

    @pl.loop(0, CHH)
    def _(i):
        @pl.loop(0, K // LANES)
        def _(q):
            v = eidx[i, pl.ds(q * LANES, LANES)]
            r = lax.shift_right_logical(v, 7)
            c = lax.bitwise_and(v, 127)
            plsc.addupdate_scatter(bins_v, [r, c], ones16)

    pltpu.sync_copy(bins_v, bins_sh.at[idv], add=True)
    plsc.subcore_barrier()

    pltpu.sync_copy(bins_sh, bins_v)

    r0 = sid * RPT

    def expand(nrows):
        @pl.loop(0, nrows)
        def _(j):
            node = r0 + j
            r16 = jnp.full((LANES,), lax.shift_right_logical(node, 7),
                           jnp.int32)
            c16 = jnp.full((LANES,), lax.bitwise_and(node, 127), jnp.int32)
            cnt16 = plsc.load_gather(bins_v, [r16, c16])

            @pl.loop(0, D // LANES)
            def _(q):
                exp_v[j, pl.ds(q * LANES, LANES)] = cnt16

    out_hbm_sel = [hs_hbm, hd_hbm]

    @pl.when(sid < NS - 1)
    def _():
        expand(RPT)

    @pl.when(sid == NS - 1)
    def _():
        expand(RPT + TAIL)

    for c in range(NC):
        @pl.when(cid == c)
        def _():
            @pl.when(sid < NS - 1)
            def _():
                pltpu.sync_copy(exp_v.at[pl.ds(0, RPT)],
                                out_hbm_sel[c].at[pl.ds(r0, RPT)])

            @pl.when(sid == NS - 1)
            def _():
                pltpu.sync_copy(exp_v,
                                out_hbm_sel[c].at[pl.ds(r0, RPT + TAIL)])


@functools.partial(
    pl.kernel,
    out_type=jax.ShapeDtypeStruct((NC * N, D), jnp.float32),
    mesh=_mesh,
    scratch_types=[
        pltpu.VMEM((CHS, K), jnp.int32),
        pltpu.VMEM((CHS, K), jnp.int32),
        pltpu.VMEM((K, D), jnp.float32),
        pltpu.VMEM((K, D), jnp.float32),
        pltpu.VMEM((K, D), jnp.float32),
        pltpu.VMEM((ZR, D), jnp.float32),
        pltpu.VMEM_SHARED((N, D), jnp.float32),
        pltpu.SemaphoreType.DMA,
        pltpu.SemaphoreType.DMA,
        pltpu.SemaphoreType.DMA,
        pltpu.SemaphoreType.DMA,
        pltpu.SemaphoreType.DMA,
        pltpu.SemaphoreType.DMA,
    ],
)
def _message_pass(feat_hbm, src_hbm, dst_hbm, acc_hbm,
                  sidx, didx, rows0, rows1, rows2, zero_v, acc_s,
                  g0, g1, g2, s0, s1, s2):
    cid = lax.axis_index("c")
    sid = lax.axis_index("s")
    w = cid * NS + sid

    @pl.loop(0, ZR)
    def _(j):
        @pl.loop(0, D // LANES)
        def _(q):
            zero_v[j, pl.ds(q * LANES, LANES)] = jnp.zeros((LANES,), jnp.float32)

    r0 = sid * RPT

    @pl.loop(0, RPT // ZR)
    def _(t):
        pltpu.sync_copy(zero_v, acc_s.at[pl.ds(r0 + t * ZR, ZR)])

    @pl.when(sid == NS - 1)
    def _():
        pltpu.sync_copy(zero_v, acc_s.at[pl.ds(NS * RPT, TAIL)])

    plsc.subcore_barrier()

    rows = (rows0, rows1, rows2)
    gsem = (g0, g1, g2)
    ssem = (s0, s1, s2)

    def wait_gather(b):
        pltpu.make_async_copy(feat_hbm.at[sidx.at[0]], rows[b], gsem[b]).wait()

    def wait_scatter(b):
        pltpu.make_async_copy(rows[b], acc_s.at[didx.at[0]], ssem[b]).wait()

    @pl.loop(0, SEG)
    def _(s):
        pltpu.sync_copy(src_hbm.at[w * SEG + s], sidx)
        pltpu.sync_copy(dst_hbm.at[w * SEG + s], didx)
        pltpu.async_copy(feat_hbm.at[sidx.at[0]], rows0, g0)
        pltpu.async_copy(feat_hbm.at[sidx.at[1]], rows1, g1)

        @pl.loop(0, CHS // 3)
        def _(t):
            for j in range(3):
                c = 3 * t + j
                b, bp, bn = j, (j + 2) % 3, (j + 2) % 3
                wait_gather(b)
                pltpu.async_copy(rows[b], acc_s.at[didx.at[c]], ssem[b],
                                 add=True)
                if j == 0:
                    @pl.when(c > 0)
                    def _():
                        wait_scatter(bp)
                else:
                    wait_scatter((j - 1) % 3)

                @pl.when(c + 2 < CHS)
                def _():
                    pltpu.async_copy(feat_hbm.at[sidx.at[c + 2]], rows[bn],
                                     gsem[bn])

        wait_gather(0)
        pltpu.async_copy(rows0, acc_s.at[didx.at[CHS - 1]], s0, add=True)
        wait_scatter(2)
        wait_scatter(0)

    plsc.subcore_barrier()
    pltpu.sync_copy(acc_s.at[pl.ds(r0, RPT)],
                    acc_hbm.at[pl.ds(cid * N + r0, RPT)])

    @pl.when(sid == NS - 1)
    def _():
        pltpu.sync_copy(acc_s.at[pl.ds(NS * RPT, TAIL)],
                        acc_hbm.at[pl.ds(cid * N + NS * RPT, TAIL)])


_ROWS = 1000


def _feat_body(x_ref, hs_ref, o_ref):
    deg = hs_ref[...]
    scale = lax.rsqrt(jnp.maximum(deg, 1.0))
    o_ref[...] = x_ref[...] * scale


_feat_call = pl.pallas_call(
    _feat_body,
    out_shape=jax.ShapeDtypeStruct((N, D), jnp.float32),
    grid=(N // _ROWS,),
    in_specs=[
        pl.BlockSpec((_ROWS, D), lambda i: (i, 0)),
        pl.BlockSpec((_ROWS, D), lambda i: (i, 0)),
    ],
    out_specs=pl.BlockSpec((_ROWS, D), lambda i: (i, 0)),
)


def _out_body(acc_ref, feat_ref, hd_ref, w1_ref, w2_ref, o_ref):
    cs = acc_ref[0] + acc_ref[1]
    deg = hd_ref[...]
    scale = lax.rsqrt(jnp.maximum(deg, 1.0))
    t = jnp.dot(cs, w1_ref[...], preferred_element_type=jnp.float32)
    t = t + jnp.dot(feat_ref[...] * cs, w2_ref[...],
                    preferred_element_type=jnp.float32)
    o_ref[...] = t * scale


_out_call = pl.pallas_call(
    _out_body,
    out_shape=jax.ShapeDtypeStruct((N, D), jnp.float32),
    grid=(N // _ROWS,),
    in_specs=[
        pl.BlockSpec((NC, _ROWS, D), lambda i: (0, i, 0)),
        pl.BlockSpec((_ROWS, D), lambda i: (i, 0)),
        pl.BlockSpec((_ROWS, D), lambda i: (i, 0)),
        pl.BlockSpec((D, D), lambda i: (0, 0)),
        pl.BlockSpec((D, D), lambda i: (0, 0)),
    ],
    out_specs=pl.BlockSpec((_ROWS, D), lambda i: (i, 0)),
)


def kernel(x, edge_index, W1, W2):
    ei = edge_index.astype(jnp.int32)
    src = ei[0]
    dst = ei[1]
    hs, hd = _degree_hist(src.reshape(NS, CHH, K), dst.reshape(NS, CHH, K))
    feat = _feat_call(x, hs)
    acc = _message_pass(feat, src.reshape(NW * SEG, CHS, K),
                        dst.reshape(NW * SEG, CHS, K)).reshape(NC, N, D)
    return _out_call(acc, feat, hd, W1, W2)


# speedup vs baseline: 13.7824x; 1.0202x over previous
import dataclasses
import functools

import jax
import jax.numpy as jnp
from jax import lax
from jax.experimental import pallas as pl
from jax.experimental.pallas import tpu as pltpu
from jax.experimental.pallas import tpu_sc as plsc

N = 10000
E = 320000
D = 128
NC, NS = 2, 16
NW = NC * NS
EPW = E // NW
K = 80
CH = EPW // K
SEG = 5
CHS = CH // SEG
RPT = 624
TAIL = N - NS * RPT
ZR = 16
LANES = 16

_mesh = plsc.VectorSubcoreMesh(core_axis_name="c", subcore_axis_name="s")
_cp = pltpu.CompilerParams()
if "needs_layout_passes" in pltpu.CompilerParams.__dataclass_fields__:
    _cp = dataclasses.replace(_cp, needs_layout_passes=False)

NB = 80
ECT = E // NS
CHH = ECT // K


@functools.partial(
    pl.kernel,
    out_type=[
        jax.ShapeDtypeStruct((N, D), jnp.float32),
        jax.ShapeDtypeStruct((N, D), jnp.float32),
    ],
    mesh=_mesh,
    compiler_params=_cp,
    scratch_types=[
        pltpu.VMEM((CHH, K), jnp.int32),
        pltpu.VMEM((NB, D), jnp.float32),
        pltpu.VMEM((NB,), jnp.int32),
        pltpu.VMEM((RPT + TAIL, D), jnp.float32),
        pltpu.VMEM_SHARED((NB, D), jnp.float32),
    ],
)
def _degree_hist(src_hbm, dst_hbm, hs_hbm, hd_hbm,
                 eidx, bins_v, idv, exp_v, bins_sh):
    cid = lax.axis_index("c")
    sid = lax.axis_index("s")

    @pl.loop(0, NB)
    def _(j):
        @pl.loop(0, D // LANES)
        def _(q):
            bins_v[j, pl.ds(q * LANES, LANES)] = jnp.zeros((LANES,),
                                                           jnp.float32)

    @pl.loop(0, NB // LANES)
    def _(j):
        idv[pl.ds(j * LANES, LANES)] = lax.iota(jnp.int32, LANES) + j * LANES

    @pl.when(sid == 0)
    def _():
        pltpu.sync_copy(bins_v, bins_sh.at[idv])

    @pl.when(cid == 0)
    def _():
        pltpu.sync_copy(src_hbm.at[sid], eidx)

    @pl.when(cid == 1)
    def _():
        pltpu.sync_copy(dst_hbm.at[sid], eidx)

    plsc.subcore_barrier()

    ones16 = jnp.full((LANES,), 1.0, jnp.float32)

    @pl.loop(0, CHH)
    def _(i):
        @pl.loop(0, K // LANES)
        def _(q):
            v = eidx[i, pl.ds(q * LANES, LANES)]
            r = lax.shift_right_logical(v, 7)
            c = lax.bitwise_and(v, 127)
            plsc.addupdate_scatter(bins_v, [r, c], ones16)

    pltpu.sync_copy(bins_v, bins_sh.at[idv], add=True)
    plsc.subcore_barrier()

    pltpu.sync_copy(bins_sh, bins_v)

    r0 = sid * RPT

    def expand(nrows):
        @pl.loop(0, nrows)
        def _(j):
            node = r0 + j
            r16 = jnp.full((LANES,), lax.shift_right_logical(node, 7),
                           jnp.int32)
            c16 = jnp.full((LANES,), lax.bitwise_and(node, 127), jnp.int32)
            cnt16 = plsc.load_gather(bins_v, [r16, c16])

            @pl.loop(0, D // LANES)
            def _(q):
                exp_v[j, pl.ds(q * LANES, LANES)] = cnt16

    out_hbm_sel = [hs_hbm, hd_hbm]

    @pl.when(sid < NS - 1)
    def _():
        expand(RPT)

    @pl.when(sid == NS - 1)
    def _():
        expand(RPT + TAIL)

    for c in range(NC):
        @pl.when(cid == c)
        def _():
            @pl.when(sid < NS - 1)
            def _():
                pltpu.sync_copy(exp_v.at[pl.ds(0, RPT)],
                                out_hbm_sel[c].at[pl.ds(r0, RPT)])

            @pl.when(sid == NS - 1)
            def _():
                pltpu.sync_copy(exp_v,
                                out_hbm_sel[c].at[pl.ds(r0, RPT + TAIL)])


@functools.partial(
    pl.kernel,
    out_type=jax.ShapeDtypeStruct((NC * N, D), jnp.float32),
    mesh=_mesh,
    scratch_types=[
        pltpu.VMEM((CHS, K), jnp.int32),
        pltpu.VMEM((CHS, K), jnp.int32),
        pltpu.VMEM((K, D), jnp.float32),
        pltpu.VMEM((K, D), jnp.float32),
        pltpu.VMEM((K, D), jnp.float32),
        pltpu.VMEM((ZR, D), jnp.float32),
        pltpu.VMEM_SHARED((N, D), jnp.float32),
        pltpu.SemaphoreType.DMA,
        pltpu.SemaphoreType.DMA,
        pltpu.SemaphoreType.DMA,
        pltpu.SemaphoreType.DMA,
        pltpu.SemaphoreType.DMA,
        pltpu.SemaphoreType.DMA,
    ],
)
def _message_pass(feat_hbm, src_hbm, dst_hbm, acc_hbm,
                  sidx, didx, rows0, rows1, rows2, zero_v, acc_s,
                  g0, g1, g2, s0, s1, s2):
    cid = lax.axis_index("c")
    sid = lax.axis_index("s")
    w = cid * NS + sid

    @pl.loop(0, ZR)
    def _(j):
        @pl.loop(0, D // LANES)
        def _(q):
            zero_v[j, pl.ds(q * LANES, LANES)] = jnp.zeros((LANES,), jnp.float32)

    r0 = sid * RPT

    @pl.loop(0, RPT // ZR)
    def _(t):
        pltpu.async_copy(zero_v, acc_s.at[pl.ds(r0 + t * ZR, ZR)], s2)

    @pl.when(sid == NS - 1)
    def _():
        pltpu.async_copy(zero_v, acc_s.at[pl.ds(NS * RPT, TAIL)], s2)

    pltpu.sync_copy(src_hbm.at[w * SEG], sidx)
    pltpu.sync_copy(dst_hbm.at[w * SEG], didx)
    pltpu.async_copy(feat_hbm.at[sidx.at[0]], rows0, g0)
    pltpu.async_copy(feat_hbm.at[sidx.at[1]], rows1, g1)

    @pl.loop(0, RPT // ZR)
    def _(t):
        pltpu.make_async_copy(zero_v, acc_s.at[pl.ds(r0, ZR)], s2).wait()

    @pl.when(sid == NS - 1)
    def _():
        pltpu.make_async_copy(zero_v, acc_s.at[pl.ds(r0, TAIL)], s2).wait()

    plsc.subcore_barrier()

    rows = (rows0, rows1, rows2)
    gsem = (g0, g1, g2)
    ssem = (s0, s1, s2)

    def wait_gather(b):
        pltpu.make_async_copy(feat_hbm.at[sidx.at[0]], rows[b], gsem[b]).wait()

    def wait_scatter(b):
        pltpu.make_async_copy(rows[b], acc_s.at[didx.at[0]], ssem[b]).wait()

    @pl.loop(0, SEG)
    def _(s):
        @pl.when(s > 0)
        def _():
            pltpu.sync_copy(src_hbm.at[w * SEG + s], sidx)
            pltpu.sync_copy(dst_hbm.at[w * SEG + s], didx)
            pltpu.async_copy(feat_hbm.at[sidx.at[0]], rows0, g0)
            pltpu.async_copy(feat_hbm.at[sidx.at[1]], rows1, g1)

        @pl.loop(0, CHS // 3)
        def _(t):
            for j in range(3):
                c = 3 * t + j
                b, bp, bn = j, (j + 2) % 3, (j + 2) % 3
                wait_gather(b)
                pltpu.async_copy(rows[b], acc_s.at[didx.at[c]], ssem[b],
                                 add=True)
                if j == 0:
                    @pl.when(c > 0)
                    def _():
                        wait_scatter(bp)
                else:
                    wait_scatter((j - 1) % 3)

                @pl.when(c + 2 < CHS)
                def _():
                    pltpu.async_copy(feat_hbm.at[sidx.at[c + 2]], rows[bn],
                                     gsem[bn])

        wait_gather(0)
        pltpu.async_copy(rows0, acc_s.at[didx.at[CHS - 1]], s0, add=True)
        wait_scatter(2)
        wait_scatter(0)

    plsc.subcore_barrier()
    pltpu.sync_copy(acc_s.at[pl.ds(r0, RPT)],
                    acc_hbm.at[pl.ds(cid * N + r0, RPT)])

    @pl.when(sid == NS - 1)
    def _():
        pltpu.sync_copy(acc_s.at[pl.ds(NS * RPT, TAIL)],
                        acc_hbm.at[pl.ds(cid * N + NS * RPT, TAIL)])


_ROWS = 1000


def _feat_body(x_ref, hs_ref, o_ref):
    deg = hs_ref[...]
    scale = lax.rsqrt(jnp.maximum(deg, 1.0))
    o_ref[...] = x_ref[...] * scale


_feat_call = pl.pallas_call(
    _feat_body,
    out_shape=jax.ShapeDtypeStruct((N, D), jnp.float32),
    grid=(N // _ROWS,),
    in_specs=[
        pl.BlockSpec((_ROWS, D), lambda i: (i, 0)),
        pl.BlockSpec((_ROWS, D), lambda i: (i, 0)),
    ],
    out_specs=pl.BlockSpec((_ROWS, D), lambda i: (i, 0)),
)


def _out_body(acc_ref, feat_ref, hd_ref, w1_ref, w2_ref, o_ref):
    cs = acc_ref[0] + acc_ref[1]
    deg = hd_ref[...]
    scale = lax.rsqrt(jnp.maximum(deg, 1.0))
    t = jnp.dot(cs, w1_ref[...], preferred_element_type=jnp.float32)
    t = t + jnp.dot(feat_ref[...] * cs, w2_ref[...],
                    preferred_element_type=jnp.float32)
    o_ref[...] = t * scale


_out_call = pl.pallas_call(
    _out_body,
    out_shape=jax.ShapeDtypeStruct((N, D), jnp.float32),
    grid=(N // _ROWS,),
    in_specs=[
        pl.BlockSpec((NC, _ROWS, D), lambda i: (0, i, 0)),
        pl.BlockSpec((_ROWS, D), lambda i: (i, 0)),
        pl.BlockSpec((_ROWS, D), lambda i: (i, 0)),
        pl.BlockSpec((D, D), lambda i: (0, 0)),
        pl.BlockSpec((D, D), lambda i: (0, 0)),
    ],
    out_specs=pl.BlockSpec((_ROWS, D), lambda i: (i, 0)),
)


def kernel(x, edge_index, W1, W2):
    ei = edge_index.astype(jnp.int32)
    src = ei[0]
    dst = ei[1]
    hs, hd = _degree_hist(src.reshape(NS, CHH, K), dst.reshape(NS, CHH, K))
    feat = _feat_call(x, hs)
    acc = _message_pass(feat, src.reshape(NW * SEG, CHS, K),
                        dst.reshape(NW * SEG, CHS, K)).reshape(NC, N, D)
    return _out_call(acc, feat, hd, W1, W2)
